# Initial kernel scaffold; baseline (speedup 1.0000x reference)
import functools

import jax
import jax.numpy as jnp
from jax import lax
from jax.experimental import pallas as pl
from jax.experimental.pallas import tpu as pltpu
from jax.experimental.pallas import tpu_sc as plsc

_E = 8
_K = 2
_D = 1024
_N = 8192
_T = 256
_NB = _K * _N // _T + _E
_NBT = _NB * _T

_SC_CORES = 2
_SC_SUBCORES = 16
_NW = _SC_CORES * _SC_SUBCORES
_CH = 32
_PW = _NBT // _NW
_PT = _N // _NW

_VSM = plsc.VectorSubcoreMesh(core_axis_name="c", subcore_axis_name="s")


def _worker_id():
    return lax.axis_index("s") * _SC_CORES + lax.axis_index("c")


@functools.partial(
    pl.kernel,
    out_type=jax.ShapeDtypeStruct((_NBT, _D), jnp.float32),
    mesh=_VSM,
    scratch_types=[
        pltpu.VMEM((_CH,), jnp.int32),
        pltpu.VMEM((_CH, _D), jnp.float32),
        pltpu.SemaphoreType.DMA,
    ],
)
def _sc_gather(src_ref, x_ref, xs_ref, idx_v, row_v, sem):
    base = _worker_id() * _PW

    def body(j, carry):
        off = base + j * _CH
        pltpu.sync_copy(src_ref.at[pl.ds(off, _CH)], idx_v)
        pltpu.async_copy(x_ref.at[idx_v], row_v, sem).wait()
        pltpu.sync_copy(row_v, xs_ref.at[pl.ds(off, _CH)])
        return carry

    lax.fori_loop(0, _PW // _CH, body, 0)


@functools.partial(
    pl.kernel,
    out_type=jax.ShapeDtypeStruct((_N, _D), jnp.float32),
    mesh=_VSM,
    scratch_types=[
        pltpu.VMEM((_CH,), jnp.int32),
        pltpu.VMEM((_CH,), jnp.int32),
        pltpu.VMEM((_CH, _D), jnp.float32),
        pltpu.VMEM((_CH, _D), jnp.float32),
        pltpu.SemaphoreType.DMA,
        pltpu.SemaphoreType.DMA,
    ],
)
def _sc_combine(d0_ref, d1_ref, ys_ref, out_ref, i0_v, i1_v, a_v, b_v, s0, s1):
    base = _worker_id() * _PT
    nvec = _D // 16

    def body(j, carry):
        off = base + j * _CH
        pltpu.sync_copy(d0_ref.at[pl.ds(off, _CH)], i0_v)
        pltpu.sync_copy(d1_ref.at[pl.ds(off, _CH)], i1_v)
        cp0 = pltpu.async_copy(ys_ref.at[i0_v], a_v, s0)
        cp1 = pltpu.async_copy(ys_ref.at[i1_v], b_v, s1)
        cp0.wait()
        cp1.wait()

        def add(p, c2):
            r = p // nvec
            col = (p % nvec) * 16
            a_v[r, pl.ds(col, 16)] = a_v[r, pl.ds(col, 16)] + b_v[r, pl.ds(col, 16)]
            return c2

        lax.fori_loop(0, _CH * nvec, add, 0)
        pltpu.sync_copy(a_v, out_ref.at[pl.ds(off, _CH)])
        return carry

    lax.fori_loop(0, _PT // _CH, body, 0)


def _mlp_body(be_ref, xs_ref, w1_ref, b1_ref, w2_ref, b2_ref, w3_ref, b3_ref,
              ws_ref, out_ref):
    b = pl.program_id(0)
    nb_used = be_ref[_NB]

    @pl.when(b < nb_used)
    def _():
        h = lax.dot_general(xs_ref[...], w1_ref[0], (((1,), (1,)), ((), ())),
                            preferred_element_type=jnp.float32)
        h = jnp.maximum(h + b1_ref[...], 0.0)
        h = lax.dot_general(h, w2_ref[0], (((1,), (1,)), ((), ())),
                            preferred_element_type=jnp.float32)
        h = jnp.maximum(h + b2_ref[...], 0.0)
        h = lax.dot_general(h, w3_ref[0], (((1,), (1,)), ((), ())),
                            preferred_element_type=jnp.float32)
        h = jnp.maximum(h + b3_ref[...], 0.0)
        out_ref[...] = h * ws_ref[...][:, 0:1]


def _mlp_call(scalars, xs, W1, b1, W2, b2, W3, b3, ws_b):
    grid_spec = pltpu.PrefetchScalarGridSpec(
        num_scalar_prefetch=1,
        grid=(_NB,),
        in_specs=[
            pl.BlockSpec((_T, _D), lambda i, be: (i, 0)),
            pl.BlockSpec((1, _D, _D), lambda i, be: (be[i], 0, 0)),
            pl.BlockSpec((1, _D), lambda i, be: (be[i], 0)),
            pl.BlockSpec((1, _D, _D), lambda i, be: (be[i], 0, 0)),
            pl.BlockSpec((1, _D), lambda i, be: (be[i], 0)),
            pl.BlockSpec((1, _D, _D), lambda i, be: (be[i], 0, 0)),
            pl.BlockSpec((1, _D), lambda i, be: (be[i], 0)),
            pl.BlockSpec((_T, 128), lambda i, be: (i, 0)),
        ],
        out_specs=pl.BlockSpec((_T, _D), lambda i, be: (i, 0)),
    )
    return pl.pallas_call(
        _mlp_body,
        grid_spec=grid_spec,
        out_shape=jax.ShapeDtypeStruct((_NBT, _D), jnp.float32),
    )(scalars, xs, W1, b1, W2, b2, W3, b3, ws_b)


def kernel(x, gate_w, gate_b, W1, b1, W2, b2, W3, b3):
    B, S, Din = x.shape
    x2 = x.reshape(_N, Din)

    gate_logits = jnp.einsum('bsd,ed->bse', x, gate_w) + gate_b
    gate_probs = jax.nn.softmax(gate_logits, axis=-1)
    _, top_idx = lax.top_k(gate_probs, _K)
    probs2 = gate_probs.reshape(_N, _E)
    top2 = top_idx.reshape(_N, _K).astype(jnp.int32)

    wflat = jnp.concatenate([probs2[:, 0], probs2[:, 1]])
    eflat = jnp.concatenate([top2[:, 0], top2[:, 1]])

    oh = (eflat[:, None] == jnp.arange(_E, dtype=jnp.int32)[None, :])
    ohi = oh.astype(jnp.int32)
    incl = jnp.cumsum(ohi, axis=0)
    rank = jnp.take_along_axis(incl, eflat[:, None], axis=1)[:, 0] - 1
    counts = incl[-1]
    be = (counts + _T - 1) // _T
    cumb = jnp.cumsum(be)
    pad_off = jnp.concatenate([jnp.zeros((1,), jnp.int32), cumb[:-1]]) * _T
    dest = pad_off[eflat] + rank
    ar = jnp.arange(_N, dtype=jnp.int32)
    src_tok = jnp.zeros((_NBT,), jnp.int32).at[dest].set(
        jnp.concatenate([ar, ar]))
    ws = jnp.zeros((_NBT,), jnp.float32).at[dest].set(wflat)
    ws_b = jnp.broadcast_to(ws[:, None], (_NBT, 128))
    block_expert = jnp.clip(
        jnp.searchsorted(cumb, jnp.arange(_NB, dtype=jnp.int32), side='right'),
        0, _E - 1).astype(jnp.int32)
    scalars = jnp.concatenate([block_expert, cumb[-1:]]).astype(jnp.int32)

    xs = _sc_gather(src_tok, x2)
    ys = _mlp_call(scalars, xs, W1, b1, W2, b2, W3, b3, ws_b)
    out2 = _sc_combine(dest[:_N], dest[_N:], ys)
    return out2.reshape(B, S, _D)


# speedup vs baseline: 1.1615x; 1.1615x over previous
import functools

import jax
import jax.numpy as jnp
from jax import lax
from jax.experimental import pallas as pl
from jax.experimental.pallas import tpu as pltpu
from jax.experimental.pallas import tpu_sc as plsc

_E = 8
_K = 2
_D = 1024
_N = 8192
_T = 256
_NB = _K * _N // _T + _E
_NBT = _NB * _T

_SC_CORES = 2
_SC_SUBCORES = 16
_NW = _SC_CORES * _SC_SUBCORES
_CH = 32
_PW = _NBT // _NW
_PT = _N // _NW

_VSM = plsc.VectorSubcoreMesh(core_axis_name="c", subcore_axis_name="s")


def _worker_id():
    return lax.axis_index("s") * _SC_CORES + lax.axis_index("c")


@functools.partial(
    pl.kernel,
    out_type=jax.ShapeDtypeStruct((_NBT, _D), jnp.float32),
    mesh=_VSM,
    scratch_types=[
        pltpu.VMEM((_CH,), jnp.int32),
        pltpu.VMEM((_CH, _D), jnp.float32),
        pltpu.SemaphoreType.DMA,
    ],
)
def _sc_gather(src_ref, x_ref, xs_ref, idx_v, row_v, sem):
    base = _worker_id() * _PW

    def body(j, carry):
        off = base + j * _CH
        pltpu.sync_copy(src_ref.at[pl.ds(off, _CH)], idx_v)
        pltpu.async_copy(x_ref.at[idx_v], row_v, sem).wait()
        pltpu.sync_copy(row_v, xs_ref.at[pl.ds(off, _CH)])
        return carry

    lax.fori_loop(0, _PW // _CH, body, 0)


@functools.partial(
    pl.kernel,
    out_type=jax.ShapeDtypeStruct((_N, _D), jnp.float32),
    mesh=_VSM,
    scratch_types=[
        pltpu.VMEM((_CH,), jnp.int32),
        pltpu.VMEM((_CH,), jnp.int32),
        pltpu.VMEM((_CH, _D), jnp.float32),
        pltpu.VMEM((_CH, _D), jnp.float32),
        pltpu.SemaphoreType.DMA,
        pltpu.SemaphoreType.DMA,
    ],
)
def _sc_combine(d0_ref, d1_ref, ys_ref, out_ref, i0_v, i1_v, a_v, b_v, s0, s1):
    base = _worker_id() * _PT
    nvec = _D // 16

    def body(j, carry):
        off = base + j * _CH
        pltpu.sync_copy(d0_ref.at[pl.ds(off, _CH)], i0_v)
        pltpu.sync_copy(d1_ref.at[pl.ds(off, _CH)], i1_v)
        cp0 = pltpu.async_copy(ys_ref.at[i0_v], a_v, s0)
        cp1 = pltpu.async_copy(ys_ref.at[i1_v], b_v, s1)
        cp0.wait()
        cp1.wait()

        def add(p, c2):
            r = p // nvec
            col = (p % nvec) * 16
            a_v[r, pl.ds(col, 16)] = a_v[r, pl.ds(col, 16)] + b_v[r, pl.ds(col, 16)]
            return c2

        lax.fori_loop(0, _CH * nvec, add, 0)
        pltpu.sync_copy(a_v, out_ref.at[pl.ds(off, _CH)])
        return carry

    lax.fori_loop(0, _PT // _CH, body, 0)


def _mlp_body(be_ref, xs_ref, w1_ref, b1_ref, w2_ref, b2_ref, w3_ref, b3_ref,
              ws_ref, out_ref):
    b = pl.program_id(0)
    nb_used = be_ref[_NB]

    @pl.when(b < nb_used)
    def _():
        h = lax.dot_general(xs_ref[...], w1_ref[0], (((1,), (1,)), ((), ())),
                            preferred_element_type=jnp.float32)
        h = jnp.maximum(h + b1_ref[0], 0.0)
        h = lax.dot_general(h, w2_ref[0], (((1,), (1,)), ((), ())),
                            preferred_element_type=jnp.float32)
        h = jnp.maximum(h + b2_ref[0], 0.0)
        h = lax.dot_general(h, w3_ref[0], (((1,), (1,)), ((), ())),
                            preferred_element_type=jnp.float32)
        h = jnp.maximum(h + b3_ref[0], 0.0)
        out_ref[...] = h * ws_ref[...][:, 0:1]


def _mlp_call(scalars, xs, W1, b1, W2, b2, W3, b3, ws_b):
    grid_spec = pltpu.PrefetchScalarGridSpec(
        num_scalar_prefetch=1,
        grid=(_NB,),
        in_specs=[
            pl.BlockSpec((_T, _D), lambda i, be: (i, 0)),
            pl.BlockSpec((1, _D, _D), lambda i, be: (be[i], 0, 0)),
            pl.BlockSpec((1, 1, _D), lambda i, be: (be[i], 0, 0)),
            pl.BlockSpec((1, _D, _D), lambda i, be: (be[i], 0, 0)),
            pl.BlockSpec((1, 1, _D), lambda i, be: (be[i], 0, 0)),
            pl.BlockSpec((1, _D, _D), lambda i, be: (be[i], 0, 0)),
            pl.BlockSpec((1, 1, _D), lambda i, be: (be[i], 0, 0)),
            pl.BlockSpec((_T, 128), lambda i, be: (i, 0)),
        ],
        out_specs=pl.BlockSpec((_T, _D), lambda i, be: (i, 0)),
    )
    return pl.pallas_call(
        _mlp_body,
        grid_spec=grid_spec,
        out_shape=jax.ShapeDtypeStruct((_NBT, _D), jnp.float32),
    )(scalars, xs, W1, b1.reshape(_E, 1, _D), W2, b2.reshape(_E, 1, _D),
      W3, b3.reshape(_E, 1, _D), ws_b)


def kernel(x, gate_w, gate_b, W1, b1, W2, b2, W3, b3):
    B, S, Din = x.shape
    x2 = x.reshape(_N, Din)

    gate_logits = jnp.einsum('bsd,ed->bse', x, gate_w) + gate_b
    gate_probs = jax.nn.softmax(gate_logits, axis=-1)
    _, top_idx = lax.top_k(gate_probs, _K)
    probs2 = gate_probs.reshape(_N, _E)
    top2 = top_idx.reshape(_N, _K).astype(jnp.int32)

    wflat = jnp.concatenate([probs2[:, 0], probs2[:, 1]])
    eflat = jnp.concatenate([top2[:, 0], top2[:, 1]])

    oh = (eflat[:, None] == jnp.arange(_E, dtype=jnp.int32)[None, :])
    ohi = oh.astype(jnp.int32)
    incl = jnp.cumsum(ohi, axis=0)
    rank = jnp.take_along_axis(incl, eflat[:, None], axis=1)[:, 0] - 1
    counts = incl[-1]
    be = (counts + _T - 1) // _T
    cumb = jnp.cumsum(be)
    pad_off = jnp.concatenate([jnp.zeros((1,), jnp.int32), cumb[:-1]]) * _T
    dest = pad_off[eflat] + rank
    ar = jnp.arange(_N, dtype=jnp.int32)
    src_tok = jnp.zeros((_NBT,), jnp.int32).at[dest].set(
        jnp.concatenate([ar, ar]))
    ws = jnp.zeros((_NBT,), jnp.float32).at[dest].set(wflat)
    ws_b = jnp.broadcast_to(ws[:, None], (_NBT, 128))
    block_expert = jnp.clip(
        jnp.searchsorted(cumb, jnp.arange(_NB, dtype=jnp.int32), side='right'),
        0, _E - 1).astype(jnp.int32)
    scalars = jnp.concatenate([block_expert, cumb[-1:]]).astype(jnp.int32)

    xs = _sc_gather(src_tok, x2)
    ys = _mlp_call(scalars, xs, W1, b1, W2, b2, W3, b3, ws_b)
    out2 = _sc_combine(dest[:_N], dest[_N:], ys)
    return out2.reshape(B, S, _D)


# speedup vs baseline: 1.3458x; 1.1586x over previous
import functools

import jax
import jax.numpy as jnp
from jax import lax
from jax.experimental import pallas as pl
from jax.experimental.pallas import tpu as pltpu
from jax.experimental.pallas import tpu_sc as plsc

_E = 8
_K = 2
_D = 1024
_N = 8192
_T = 256
_NB = _K * _N // _T + _E
_NBT = _NB * _T

_SC_CORES = 2
_SC_SUBCORES = 16
_NW = _SC_CORES * _SC_SUBCORES
_CH = 32
_PW = _NBT // _NW
_PT = _N // _NW

_VSM = plsc.VectorSubcoreMesh(core_axis_name="c", subcore_axis_name="s")


def _worker_id():
    return lax.axis_index("s") * _SC_CORES + lax.axis_index("c")


_GR = 3
_GCH = _PW // _CH


@functools.partial(
    pl.kernel,
    out_type=jax.ShapeDtypeStruct((_NBT, _D), jnp.float32),
    mesh=_VSM,
    scratch_types=[
        pltpu.VMEM((_PW,), jnp.int32),
        [pltpu.VMEM((_CH, _D), jnp.float32)] * _GR,
        [pltpu.SemaphoreType.DMA] * _GR,
        [pltpu.SemaphoreType.DMA] * _GR,
    ],
)
def _sc_gather(src_ref, x_ref, xs_ref, idx_v, bufs, gsem, wsem):
    base = _worker_id() * _PW
    pltpu.sync_copy(src_ref.at[pl.ds(base, _PW)], idx_v)

    def _idx(c):
        return idx_v.at[pl.ds(c * _CH, _CH)]

    for s in range(_GR):
        pltpu.async_copy(x_ref.at[_idx(s)], bufs[s], gsem[s])

    def round_body(i, carry):
        for s in range(_GR):
            c = i * _GR + s
            pltpu.make_async_copy(x_ref.at[_idx(c)], bufs[s], gsem[s]).wait()
            pltpu.async_copy(bufs[s], xs_ref.at[pl.ds(base + c * _CH, _CH)],
                             wsem[s])

            @pl.when(i < _GCH // _GR - 1)
            def _():
                pltpu.make_async_copy(
                    bufs[s], xs_ref.at[pl.ds(base + c * _CH, _CH)],
                    wsem[s]).wait()
                pltpu.async_copy(x_ref.at[_idx(c + _GR)], bufs[s], gsem[s])
        return carry

    lax.fori_loop(0, _GCH // _GR, round_body, 0)
    for s in range(_GR):
        pltpu.make_async_copy(
            bufs[s], xs_ref.at[pl.ds(base, _CH)], wsem[s]).wait()


_CC = 16
_CR = 2
_CCH = _PT // _CC
_NV = _D // 16


@functools.partial(
    pl.kernel,
    out_type=jax.ShapeDtypeStruct((_N, _D), jnp.float32),
    mesh=_VSM,
    scratch_types=[
        pltpu.VMEM((_PT,), jnp.int32),
        pltpu.VMEM((_PT,), jnp.int32),
        [pltpu.VMEM((_CC, _D), jnp.float32)] * _CR,
        [pltpu.VMEM((_CC, _D), jnp.float32)] * _CR,
        [pltpu.SemaphoreType.DMA] * _CR,
        [pltpu.SemaphoreType.DMA] * _CR,
        [pltpu.SemaphoreType.DMA] * _CR,
    ],
)
def _sc_combine(d0_ref, d1_ref, ys_ref, out_ref, i0_v, i1_v, av, bv,
                gas, gbs, wos):
    base = _worker_id() * _PT
    pltpu.sync_copy(d0_ref.at[pl.ds(base, _PT)], i0_v)
    pltpu.sync_copy(d1_ref.at[pl.ds(base, _PT)], i1_v)

    def _i0(c):
        return i0_v.at[pl.ds(c * _CC, _CC)]

    def _i1(c):
        return i1_v.at[pl.ds(c * _CC, _CC)]

    for s in range(_CR):
        pltpu.async_copy(ys_ref.at[_i0(s)], av[s], gas[s])
        pltpu.async_copy(ys_ref.at[_i1(s)], bv[s], gbs[s])

    def round_body(i, carry):
        for s in range(_CR):
            c = i * _CR + s
            pltpu.make_async_copy(ys_ref.at[_i0(c)], av[s], gas[s]).wait()
            pltpu.make_async_copy(ys_ref.at[_i1(c)], bv[s], gbs[s]).wait()

            def add_row(r, c2):
                for v in range(_NV):
                    av[s][r, pl.ds(v * 16, 16)] = (
                        av[s][r, pl.ds(v * 16, 16)]
                        + bv[s][r, pl.ds(v * 16, 16)])
                return c2

            lax.fori_loop(0, _CC, add_row, 0)
            pltpu.async_copy(av[s], out_ref.at[pl.ds(base + c * _CC, _CC)],
                             wos[s])

            @pl.when(i < _CCH // _CR - 1)
            def _():
                pltpu.async_copy(ys_ref.at[_i1(c + _CR)], bv[s], gbs[s])
                pltpu.make_async_copy(
                    av[s], out_ref.at[pl.ds(base, _CC)], wos[s]).wait()
                pltpu.async_copy(ys_ref.at[_i0(c + _CR)], av[s], gas[s])
        return carry

    lax.fori_loop(0, _CCH // _CR, round_body, 0)
    for s in range(_CR):
        pltpu.make_async_copy(
            av[s], out_ref.at[pl.ds(base, _CC)], wos[s]).wait()


def _mlp_body(be_ref, xs_ref, w1_ref, b1_ref, w2_ref, b2_ref, w3_ref, b3_ref,
              ws_ref, out_ref):
    b = pl.program_id(0)
    nb_used = be_ref[_NB]

    @pl.when(b < nb_used)
    def _():
        h = lax.dot_general(xs_ref[...], w1_ref[0], (((1,), (1,)), ((), ())),
                            preferred_element_type=jnp.float32)
        h = jnp.maximum(h + b1_ref[0], 0.0)
        h = lax.dot_general(h, w2_ref[0], (((1,), (1,)), ((), ())),
                            preferred_element_type=jnp.float32)
        h = jnp.maximum(h + b2_ref[0], 0.0)
        h = lax.dot_general(h, w3_ref[0], (((1,), (1,)), ((), ())),
                            preferred_element_type=jnp.float32)
        h = jnp.maximum(h + b3_ref[0], 0.0)
        out_ref[...] = h * ws_ref[...][:, 0:1]


def _mlp_call(scalars, xs, W1, b1, W2, b2, W3, b3, ws_b):
    grid_spec = pltpu.PrefetchScalarGridSpec(
        num_scalar_prefetch=1,
        grid=(_NB,),
        in_specs=[
            pl.BlockSpec((_T, _D), lambda i, be: (i, 0)),
            pl.BlockSpec((1, _D, _D), lambda i, be: (be[i], 0, 0)),
            pl.BlockSpec((1, 1, _D), lambda i, be: (be[i], 0, 0)),
            pl.BlockSpec((1, _D, _D), lambda i, be: (be[i], 0, 0)),
            pl.BlockSpec((1, 1, _D), lambda i, be: (be[i], 0, 0)),
            pl.BlockSpec((1, _D, _D), lambda i, be: (be[i], 0, 0)),
            pl.BlockSpec((1, 1, _D), lambda i, be: (be[i], 0, 0)),
            pl.BlockSpec((_T, 128), lambda i, be: (i, 0)),
        ],
        out_specs=pl.BlockSpec((_T, _D), lambda i, be: (i, 0)),
    )
    return pl.pallas_call(
        _mlp_body,
        grid_spec=grid_spec,
        out_shape=jax.ShapeDtypeStruct((_NBT, _D), jnp.float32),
    )(scalars, xs, W1, b1.reshape(_E, 1, _D), W2, b2.reshape(_E, 1, _D),
      W3, b3.reshape(_E, 1, _D), ws_b)


def kernel(x, gate_w, gate_b, W1, b1, W2, b2, W3, b3):
    B, S, Din = x.shape
    x2 = x.reshape(_N, Din)

    gate_logits = jnp.einsum('bsd,ed->bse', x, gate_w) + gate_b
    gate_probs = jax.nn.softmax(gate_logits, axis=-1)
    _, top_idx = lax.top_k(gate_probs, _K)
    probs2 = gate_probs.reshape(_N, _E)
    top2 = top_idx.reshape(_N, _K).astype(jnp.int32)

    wflat = jnp.concatenate([probs2[:, 0], probs2[:, 1]])
    eflat = jnp.concatenate([top2[:, 0], top2[:, 1]])

    oh = (eflat[:, None] == jnp.arange(_E, dtype=jnp.int32)[None, :])
    ohi = oh.astype(jnp.int32)
    incl = jnp.cumsum(ohi, axis=0)
    rank = jnp.take_along_axis(incl, eflat[:, None], axis=1)[:, 0] - 1
    counts = incl[-1]
    be = (counts + _T - 1) // _T
    cumb = jnp.cumsum(be)
    pad_off = jnp.concatenate([jnp.zeros((1,), jnp.int32), cumb[:-1]]) * _T
    dest = pad_off[eflat] + rank
    ar = jnp.arange(_N, dtype=jnp.int32)
    src_tok = jnp.zeros((_NBT,), jnp.int32).at[dest].set(
        jnp.concatenate([ar, ar]))
    ws = jnp.zeros((_NBT,), jnp.float32).at[dest].set(wflat)
    ws_b = jnp.broadcast_to(ws[:, None], (_NBT, 128))
    block_expert = jnp.clip(
        jnp.searchsorted(cumb, jnp.arange(_NB, dtype=jnp.int32), side='right'),
        0, _E - 1).astype(jnp.int32)
    scalars = jnp.concatenate([block_expert, cumb[-1:]]).astype(jnp.int32)

    xs = _sc_gather(src_tok, x2)
    ys = _mlp_call(scalars, xs, W1, b1, W2, b2, W3, b3, ws_b)
    out2 = _sc_combine(dest[:_N], dest[_N:], ys)
    return out2.reshape(B, S, _D)
